# baseline (speedup 1.0000x reference)
import functools

import jax
import jax.numpy as jnp
import numpy as np
from jax import lax
from jax.experimental import pallas as pl
from jax.experimental.pallas import tpu as pltpu
from jax.experimental.pallas import tpu_sc as plsc

NQB = 2
NH = 8
NP = 4
D = 256
DH = 32
SH = 128
SW = 128
Q = SH * SW
J = NP * NH * NQB
NSLOT = 4 * J

T1 = 2048

NWORK = 32
QPW = Q // NWORK
CQ = 8
NCHUNK = QPW // CQ
CROWS = CQ * NSLOT
OUTB = 8


def _prep_body(q_ref, rx_ref, ry_ref, wvt_ref, bv_ref, wsx_ref, bsx_ref,
               wsy_ref, bsy_ref, waw_ref, baw_ref, vt_ref, idx_ref, wgt_ref):
    q = q_ref[...]
    vt_ref[...] = (jnp.dot(q, wvt_ref[...], preferred_element_type=jnp.float32)
                   + bv_ref[...]).astype(jnp.bfloat16)
    sox = jnp.dot(q, wsx_ref[...], preferred_element_type=jnp.float32) + bsx_ref[...]
    soy = jnp.dot(q, wsy_ref[...], preferred_element_type=jnp.float32) + bsy_ref[...]
    aw = jnp.dot(q, waw_ref[...], preferred_element_type=jnp.float32) + baw_ref[...]

    a0, a1, a2, a3 = (aw[:, 0:16], aw[:, 16:32], aw[:, 32:48], aw[:, 48:64])
    m = jnp.maximum(jnp.maximum(a0, a1), jnp.maximum(a2, a3))
    e0, e1, e2, e3 = (jnp.exp(a0 - m), jnp.exp(a1 - m), jnp.exp(a2 - m),
                      jnp.exp(a3 - m))
    inv = 1.0 / (e0 + e1 + e2 + e3)
    w64 = jnp.concatenate([e0 * inv, e1 * inv, e2 * inv, e3 * inv], axis=1)

    rx64 = jnp.concatenate([rx_ref[...]] * 4, axis=1)
    ry64 = jnp.concatenate([ry_ref[...]] * 4, axis=1)

    locx = rx64 + sox * (1.0 / SW)
    locy = ry64 + soy * (1.0 / SH)
    gx = ((2.0 * locx - 1.0) + 1.0) * (SW / 2.0) - 0.5
    gy = ((2.0 * locy - 1.0) + 1.0) * (SH / 2.0) - 0.5
    x0 = jnp.floor(gx)
    y0 = jnp.floor(gy)
    wx1 = gx - x0
    wx0 = 1.0 - wx1
    wy1 = gy - y0
    wy0 = 1.0 - wy1

    hcol = (lax.broadcasted_iota(jnp.int32, (T1, J), 1) % 16) // 2
    corners = ((0.0, 0.0, wx0, wy0), (1.0, 0.0, wx1, wy0),
               (0.0, 1.0, wx0, wy1), (1.0, 1.0, wx1, wy1))
    for c, (dx, dy, wx, wy) in enumerate(corners):
        xi = x0 + dx
        yi = y0 + dy
        valid = ((xi >= 0.0) & (xi <= SW - 1.0)
                 & (yi >= 0.0) & (yi <= SH - 1.0))
        xc = jnp.clip(xi, 0.0, SW - 1.0).astype(jnp.int32)
        yc = jnp.clip(yi, 0.0, SH - 1.0).astype(jnp.int32)
        idx_ref[:, c * J:(c + 1) * J] = (yc * SW + xc) * NH + hcol
        wgt_ref[:, c * J:(c + 1) * J] = (wx * wy * w64 * 0.5
                                         * valid.astype(jnp.float32))


def _out_body(s_ref, q_ref, wot_ref, bo_ref, o_ref):
    o_ref[...] = (jnp.dot(s_ref[...], wot_ref[...],
                          preferred_element_type=jnp.float32)
                  + bo_ref[...] + q_ref[...])


HCR = CROWS // 2


def _sc_body(vtab, idxh, wgth, outh,
             idx0, idx1, idx2, idx3, wgt0, wgt1, wgt2, wgt3,
             rows0, rows1, out_v,
             sga0, sga1, sgb0, sgb1,
             six0, six1, six2, six3, swg0, swg1, swg2, swg3, so):
    wid = lax.axis_index("s") * 2 + lax.axis_index("c")
    base = wid * (QPW * NSLOT)
    idxs = (idx0, idx1, idx2, idx3)
    wgts = (wgt0, wgt1, wgt2, wgt3)
    rows = (rows0, rows1)
    sgas, sgbs = (sga0, sga1), (sgb0, sgb1)
    sixs = (six0, six1, six2, six3)
    swgs = (swg0, swg1, swg2, swg3)

    def fetch(i, m):
        off = base + i * CROWS
        pltpu.async_copy(idxh.at[pl.ds(off, CROWS)], idxs[m], sixs[m])
        pltpu.async_copy(wgth.at[pl.ds(off, CROWS)], wgts[m], swgs[m])

    def fire(i, m, par):
        off = base + i * CROWS
        pltpu.make_async_copy(idxh.at[pl.ds(off, CROWS)], idxs[m], sixs[m]).wait()
        pltpu.make_async_copy(wgth.at[pl.ds(off, CROWS)], wgts[m], swgs[m]).wait()
        pltpu.async_copy(vtab.at[idxs[m].at[pl.ds(0, HCR)]],
                         rows[par].at[pl.ds(0, HCR)], sgas[par])
        pltpu.async_copy(vtab.at[idxs[m].at[pl.ds(HCR, HCR)]],
                         rows[par].at[pl.ds(HCR, HCR)], sgbs[par])

    fetch(0, 0)
    fetch(1, 1)
    fetch(2, 2)
    fetch(3, 3)
    fire(0, 0, 0)
    fire(1, 1, 1)

    def quad(i4, carry):
        for m in range(4):
            par = m % 2
            i = i4 * 4 + m
            if m == 0:
                @pl.when((lax.rem(i4, 2) == 0) & (i4 >= 2))
                def _wait_out():
                    pltpu.make_async_copy(
                        out_v, outh.at[pl.ds(base, OUTB * CROWS)], so).wait()

            pltpu.make_async_copy(vtab.at[idxs[m].at[pl.ds(0, HCR)]],
                                  rows[par].at[pl.ds(0, HCR)], sgas[par]).wait()

            obase = lax.rem(i, OUTB) * CROWS

            def qloop(qi, qcarry):
                rbase = qi * NSLOT
                acc = [[jnp.zeros((16,), jnp.float32) for _ in range(2)]
                       for _ in range(NH)]
                for g in range(NSLOT // 16):
                    wv = wgts[m][pl.ds(rbase + g * 16, 16)]
                    for k in range(16):
                        j = g * 16 + k
                        h = (j % 16) // 2
                        w = wv[k]
                        row = rows[par][rbase + j]
                        re, ro = plsc.unpack(
                            row, format=plsc.PackFormat.INTERLEAVED)
                        acc[h][0] = acc[h][0] + w * re
                        acc[h][1] = acc[h][1] + w * ro
                for h in range(NH):
                    out_v[pl.ds(obase + rbase + h * DH, 16)] = acc[h][0]
                    out_v[pl.ds(obase + rbase + h * DH + 16, 16)] = acc[h][1]
                return qcarry

            lax.fori_loop(0, CQ // 2, qloop, 0)
            pltpu.make_async_copy(vtab.at[idxs[m].at[pl.ds(HCR, HCR)]],
                                  rows[par].at[pl.ds(HCR, HCR)], sgbs[par]).wait()
            lax.fori_loop(CQ // 2, CQ, qloop, 0)

            @pl.when(i + 4 < NCHUNK)
            def _prefetch():
                fetch(i + 4, m)

            @pl.when(i + 2 < NCHUNK)
            def _fire():
                fire(i + 2, (m + 2) % 4, par)

            if m == 3:
                @pl.when(lax.rem(i4, 2) == 1)
                def _flush():
                    off0 = base + (i - (OUTB - 1)) * CROWS
                    pltpu.async_copy(out_v, outh.at[pl.ds(off0, OUTB * CROWS)], so)
        return carry

    lax.fori_loop(0, NCHUNK // 4, quad, 0)
    pltpu.make_async_copy(out_v, outh.at[pl.ds(base, OUTB * CROWS)], so).wait()


def kernel(query, key, value, reference_points, spatial_shapes,
           level_start_index, Wso, bso, Waw, baw, Wv, bv, Wo, bo):
    del key, value, spatial_shapes, level_start_index
    q2d = query.reshape(Q, D)

    Wso_c = Wso[:, :D] + Wso[:, D:]
    Waw_c = Waw[:, :D] + Waw[:, D:]

    so_x_rows, so_y_rows, aw_rows = [], [], []
    for p in range(NP):
        for h in range(NH):
            for bq in range(NQB):
                so_x_rows.append(h * 16 + bq * 8 + p * 2 + 0)
                so_y_rows.append(h * 16 + bq * 8 + p * 2 + 1)
                aw_rows.append(h * 8 + bq * 4 + p)
    so_x_rows = np.array(so_x_rows)
    so_y_rows = np.array(so_y_rows)
    aw_rows = np.array(aw_rows)

    wvt = Wv.T
    wsx = Wso_c[so_x_rows].T
    wsy = Wso_c[so_y_rows].T
    waw = Waw_c[aw_rows].T
    bsx = bso[so_x_rows].reshape(1, J)
    bsy = bso[so_y_rows].reshape(1, J)
    bawp = baw[aw_rows].reshape(1, J)
    bv2 = bv.reshape(1, D)
    bo2 = bo.reshape(1, D)

    rx = reference_points[:, :, 0, 0]
    ry = reference_points[:, :, 0, 1]
    rx16 = jnp.tile(rx.T, (1, 8))
    ry16 = jnp.tile(ry.T, (1, 8))

    dh_perm = np.empty((D,), np.int64)
    for h in range(NH):
        for k in range(16):
            dh_perm[h * DH + k] = h * DH + 2 * k
            dh_perm[h * DH + 16 + k] = h * DH + 2 * k + 1

    grid = (Q // T1,)
    row_spec = lambda w: pl.BlockSpec((T1, w), lambda i: (i, 0))
    full_spec = lambda a, b: pl.BlockSpec((a, b), lambda i: (0, 0))

    vt, idx, wgt = pl.pallas_call(
        _prep_body,
        grid=grid,
        in_specs=[row_spec(D), row_spec(16), row_spec(16),
                  full_spec(D, D), full_spec(1, D),
                  full_spec(D, J), full_spec(1, J),
                  full_spec(D, J), full_spec(1, J),
                  full_spec(D, J), full_spec(1, J)],
        out_specs=[row_spec(D), row_spec(NSLOT), row_spec(NSLOT)],
        out_shape=[jax.ShapeDtypeStruct((Q, D), jnp.bfloat16),
                   jax.ShapeDtypeStruct((Q, NSLOT), jnp.int32),
                   jax.ShapeDtypeStruct((Q, NSLOT), jnp.float32)],
    )(q2d, rx16, ry16, wvt, bv2, wsx, bsx, wsy, bsy, waw, bawp)

    mesh = plsc.VectorSubcoreMesh(core_axis_name="c", subcore_axis_name="s")
    sam_flat = pl.kernel(
        _sc_body,
        mesh=mesh,
        compiler_params=pltpu.CompilerParams(use_tc_tiling_on_sc=False,
                                             needs_layout_passes=False),
        out_type=jax.ShapeDtypeStruct((Q * D,), jnp.float32),
        scratch_types=[
            pltpu.VMEM((CROWS,), jnp.int32),
            pltpu.VMEM((CROWS,), jnp.int32),
            pltpu.VMEM((CROWS,), jnp.int32),
            pltpu.VMEM((CROWS,), jnp.int32),
            pltpu.VMEM((CROWS,), jnp.float32),
            pltpu.VMEM((CROWS,), jnp.float32),
            pltpu.VMEM((CROWS,), jnp.float32),
            pltpu.VMEM((CROWS,), jnp.float32),
            pltpu.VMEM((CROWS, DH), jnp.bfloat16),
            pltpu.VMEM((CROWS, DH), jnp.bfloat16),
            pltpu.VMEM((OUTB * CROWS,), jnp.float32),
            pltpu.SemaphoreType.DMA,
            pltpu.SemaphoreType.DMA,
            pltpu.SemaphoreType.DMA,
            pltpu.SemaphoreType.DMA,
            pltpu.SemaphoreType.DMA,
            pltpu.SemaphoreType.DMA,
            pltpu.SemaphoreType.DMA,
            pltpu.SemaphoreType.DMA,
            pltpu.SemaphoreType.DMA,
            pltpu.SemaphoreType.DMA,
            pltpu.SemaphoreType.DMA,
            pltpu.SemaphoreType.DMA,
            pltpu.SemaphoreType.DMA,
        ],
    )(vt.reshape(Q * NH, DH), idx.reshape(Q * NSLOT), wgt.reshape(Q * NSLOT))

    sam = sam_flat.reshape(Q, D)

    out = pl.pallas_call(
        _out_body,
        grid=grid,
        in_specs=[row_spec(D), row_spec(D), full_spec(D, D), full_spec(1, D)],
        out_specs=row_spec(D),
        out_shape=jax.ShapeDtypeStruct((Q, D), jnp.float32),
    )(sam, q2d, Wo.T[dh_perm], bo2)

    return out.reshape(1, Q, D)


# speedup vs baseline: 1.1029x; 1.1029x over previous
import functools

import jax
import jax.numpy as jnp
import numpy as np
from jax import lax
from jax.experimental import pallas as pl
from jax.experimental.pallas import tpu as pltpu
from jax.experimental.pallas import tpu_sc as plsc

NQB = 2
NH = 8
NP = 4
D = 256
DH = 32
SH = 128
SW = 128
Q = SH * SW
J = NP * NH * NQB
NSLOT = 4 * J

T1 = 2048

NWORK = 32
QPW = Q // NWORK
CQ = 8
NCHUNK = QPW // CQ
CROWS = CQ * NSLOT
OUTB = 8


def _prep_body(q_ref, rx_ref, ry_ref, wvt_ref, bv_ref, wsx_ref, bsx_ref,
               wsy_ref, bsy_ref, waw_ref, baw_ref, vt_ref, idx_ref, wgt_ref):
    q = q_ref[...]
    vt_ref[...] = (jnp.dot(q, wvt_ref[...], preferred_element_type=jnp.float32)
                   + bv_ref[...]).astype(jnp.bfloat16)
    sox = jnp.dot(q, wsx_ref[...], preferred_element_type=jnp.float32) + bsx_ref[...]
    soy = jnp.dot(q, wsy_ref[...], preferred_element_type=jnp.float32) + bsy_ref[...]
    aw = jnp.dot(q, waw_ref[...], preferred_element_type=jnp.float32) + baw_ref[...]

    a0, a1, a2, a3 = (aw[:, 0:16], aw[:, 16:32], aw[:, 32:48], aw[:, 48:64])
    m = jnp.maximum(jnp.maximum(a0, a1), jnp.maximum(a2, a3))
    e0, e1, e2, e3 = (jnp.exp(a0 - m), jnp.exp(a1 - m), jnp.exp(a2 - m),
                      jnp.exp(a3 - m))
    inv = 1.0 / (e0 + e1 + e2 + e3)
    w64 = jnp.concatenate([e0 * inv, e1 * inv, e2 * inv, e3 * inv], axis=1)

    rx64 = jnp.concatenate([rx_ref[...]] * 4, axis=1)
    ry64 = jnp.concatenate([ry_ref[...]] * 4, axis=1)

    locx = rx64 + sox * (1.0 / SW)
    locy = ry64 + soy * (1.0 / SH)
    gx = ((2.0 * locx - 1.0) + 1.0) * (SW / 2.0) - 0.5
    gy = ((2.0 * locy - 1.0) + 1.0) * (SH / 2.0) - 0.5
    x0 = jnp.floor(gx)
    y0 = jnp.floor(gy)
    wx1 = gx - x0
    wx0 = 1.0 - wx1
    wy1 = gy - y0
    wy0 = 1.0 - wy1

    hcol = (lax.broadcasted_iota(jnp.int32, (T1, J), 1) % 16) // 2
    corners = ((0.0, 0.0, wx0, wy0), (1.0, 0.0, wx1, wy0),
               (0.0, 1.0, wx0, wy1), (1.0, 1.0, wx1, wy1))
    for c, (dx, dy, wx, wy) in enumerate(corners):
        xi = x0 + dx
        yi = y0 + dy
        valid = ((xi >= 0.0) & (xi <= SW - 1.0)
                 & (yi >= 0.0) & (yi <= SH - 1.0))
        xc = jnp.clip(xi, 0.0, SW - 1.0).astype(jnp.int32)
        yc = jnp.clip(yi, 0.0, SH - 1.0).astype(jnp.int32)
        idx_ref[:, c * J:(c + 1) * J] = (yc * SW + xc) * NH + hcol
        wgt_ref[:, c * J:(c + 1) * J] = (wx * wy * w64 * 0.5
                                         * valid.astype(jnp.float32))


def _out_body(s_ref, q_ref, wot_ref, bo_ref, o_ref):
    o_ref[...] = (jnp.dot(s_ref[...], wot_ref[...],
                          preferred_element_type=jnp.float32)
                  + bo_ref[...] + q_ref[...])


def _sc_body(vtab, idxh, wgth, outh,
             idx0, idx1, idx2, idx3, wgt0, wgt1, wgt2, wgt3,
             rows0, rows1, out_v,
             sga0, sga1,
             six0, six1, six2, six3, swg0, swg1, swg2, swg3, so):
    wid = lax.axis_index("s") * 2 + lax.axis_index("c")
    base = wid * (QPW * NSLOT)
    idxs = (idx0, idx1, idx2, idx3)
    wgts = (wgt0, wgt1, wgt2, wgt3)
    rows = (rows0, rows1)
    sgas = (sga0, sga1)
    sixs = (six0, six1, six2, six3)
    swgs = (swg0, swg1, swg2, swg3)

    def fetch(i, m):
        off = base + i * CROWS
        pltpu.async_copy(idxh.at[pl.ds(off, CROWS)], idxs[m], sixs[m])
        pltpu.async_copy(wgth.at[pl.ds(off, CROWS)], wgts[m], swgs[m])

    def fire(i, m, par):
        off = base + i * CROWS
        pltpu.make_async_copy(idxh.at[pl.ds(off, CROWS)], idxs[m], sixs[m]).wait()
        pltpu.make_async_copy(wgth.at[pl.ds(off, CROWS)], wgts[m], swgs[m]).wait()
        pltpu.async_copy(vtab.at[idxs[m]], rows[par], sgas[par])

    fetch(0, 0)
    fetch(1, 1)
    fetch(2, 2)
    fetch(3, 3)
    fire(0, 0, 0)
    fire(1, 1, 1)

    def quad(i4, carry):
        for m in range(4):
            par = m % 2
            i = i4 * 4 + m
            if m == 0:
                @pl.when((lax.rem(i4, 2) == 0) & (i4 >= 2))
                def _wait_out():
                    pltpu.make_async_copy(
                        out_v, outh.at[pl.ds(base, OUTB * CROWS)], so).wait()

            pltpu.make_async_copy(vtab.at[idxs[m]], rows[par], sgas[par]).wait()

            obase = lax.rem(i, OUTB) * CROWS

            def qloop(qi, qcarry):
                rbase = qi * NSLOT
                acc = [[jnp.zeros((16,), jnp.float32) for _ in range(2)]
                       for _ in range(NH)]
                for g in range(NSLOT // 16):
                    wv = wgts[m][pl.ds(rbase + g * 16, 16)]
                    for k in range(16):
                        j = g * 16 + k
                        h = (j % 16) // 2
                        w = wv[k]
                        row = rows[par][rbase + j]
                        re, ro = plsc.unpack(
                            row, format=plsc.PackFormat.INTERLEAVED)
                        acc[h][0] = acc[h][0] + w * re
                        acc[h][1] = acc[h][1] + w * ro
                for h in range(NH):
                    out_v[pl.ds(obase + rbase + h * DH, 16)] = acc[h][0]
                    out_v[pl.ds(obase + rbase + h * DH + 16, 16)] = acc[h][1]
                return qcarry

            lax.fori_loop(0, CQ, qloop, 0)

            @pl.when(i + 4 < NCHUNK)
            def _prefetch():
                fetch(i + 4, m)

            @pl.when(i + 2 < NCHUNK)
            def _fire():
                fire(i + 2, (m + 2) % 4, par)

            if m == 3:
                @pl.when(lax.rem(i4, 2) == 1)
                def _flush():
                    off0 = base + (i - (OUTB - 1)) * CROWS
                    pltpu.async_copy(out_v, outh.at[pl.ds(off0, OUTB * CROWS)], so)
        return carry

    lax.fori_loop(0, NCHUNK // 4, quad, 0)
    pltpu.make_async_copy(out_v, outh.at[pl.ds(base, OUTB * CROWS)], so).wait()


def kernel(query, key, value, reference_points, spatial_shapes,
           level_start_index, Wso, bso, Waw, baw, Wv, bv, Wo, bo):
    del key, value, spatial_shapes, level_start_index
    q2d = query.reshape(Q, D)

    Wso_c = Wso[:, :D] + Wso[:, D:]
    Waw_c = Waw[:, :D] + Waw[:, D:]

    so_x_rows, so_y_rows, aw_rows = [], [], []
    for p in range(NP):
        for h in range(NH):
            for bq in range(NQB):
                so_x_rows.append(h * 16 + bq * 8 + p * 2 + 0)
                so_y_rows.append(h * 16 + bq * 8 + p * 2 + 1)
                aw_rows.append(h * 8 + bq * 4 + p)
    so_x_rows = np.array(so_x_rows)
    so_y_rows = np.array(so_y_rows)
    aw_rows = np.array(aw_rows)

    wvt = Wv.T
    wsx = Wso_c[so_x_rows].T
    wsy = Wso_c[so_y_rows].T
    waw = Waw_c[aw_rows].T
    bsx = bso[so_x_rows].reshape(1, J)
    bsy = bso[so_y_rows].reshape(1, J)
    bawp = baw[aw_rows].reshape(1, J)
    bv2 = bv.reshape(1, D)
    bo2 = bo.reshape(1, D)

    rx = reference_points[:, :, 0, 0]
    ry = reference_points[:, :, 0, 1]
    rx16 = jnp.tile(rx.T, (1, 8))
    ry16 = jnp.tile(ry.T, (1, 8))

    dh_perm = np.empty((D,), np.int64)
    for h in range(NH):
        for k in range(16):
            dh_perm[h * DH + k] = h * DH + 2 * k
            dh_perm[h * DH + 16 + k] = h * DH + 2 * k + 1

    grid = (Q // T1,)
    row_spec = lambda w: pl.BlockSpec((T1, w), lambda i: (i, 0))
    full_spec = lambda a, b: pl.BlockSpec((a, b), lambda i: (0, 0))

    vt, idx, wgt = pl.pallas_call(
        _prep_body,
        grid=grid,
        in_specs=[row_spec(D), row_spec(16), row_spec(16),
                  full_spec(D, D), full_spec(1, D),
                  full_spec(D, J), full_spec(1, J),
                  full_spec(D, J), full_spec(1, J),
                  full_spec(D, J), full_spec(1, J)],
        out_specs=[row_spec(D), row_spec(NSLOT), row_spec(NSLOT)],
        out_shape=[jax.ShapeDtypeStruct((Q, D), jnp.bfloat16),
                   jax.ShapeDtypeStruct((Q, NSLOT), jnp.int32),
                   jax.ShapeDtypeStruct((Q, NSLOT), jnp.float32)],
    )(q2d, rx16, ry16, wvt, bv2, wsx, bsx, wsy, bsy, waw, bawp)

    mesh = plsc.VectorSubcoreMesh(core_axis_name="c", subcore_axis_name="s")
    sam_flat = pl.kernel(
        _sc_body,
        mesh=mesh,
        compiler_params=pltpu.CompilerParams(use_tc_tiling_on_sc=False,
                                             needs_layout_passes=False),
        out_type=jax.ShapeDtypeStruct((Q * D,), jnp.float32),
        scratch_types=[
            pltpu.VMEM((CROWS,), jnp.int32),
            pltpu.VMEM((CROWS,), jnp.int32),
            pltpu.VMEM((CROWS,), jnp.int32),
            pltpu.VMEM((CROWS,), jnp.int32),
            pltpu.VMEM((CROWS,), jnp.float32),
            pltpu.VMEM((CROWS,), jnp.float32),
            pltpu.VMEM((CROWS,), jnp.float32),
            pltpu.VMEM((CROWS,), jnp.float32),
            pltpu.VMEM((CROWS, DH), jnp.bfloat16),
            pltpu.VMEM((CROWS, DH), jnp.bfloat16),
            pltpu.VMEM((OUTB * CROWS,), jnp.float32),
            pltpu.SemaphoreType.DMA,
            pltpu.SemaphoreType.DMA,
            pltpu.SemaphoreType.DMA,
            pltpu.SemaphoreType.DMA,
            pltpu.SemaphoreType.DMA,
            pltpu.SemaphoreType.DMA,
            pltpu.SemaphoreType.DMA,
            pltpu.SemaphoreType.DMA,
            pltpu.SemaphoreType.DMA,
            pltpu.SemaphoreType.DMA,
            pltpu.SemaphoreType.DMA,
        ],
    )(vt.reshape(Q * NH, DH), idx.reshape(Q * NSLOT), wgt.reshape(Q * NSLOT))

    sam = sam_flat.reshape(Q, D)

    out = pl.pallas_call(
        _out_body,
        grid=grid,
        in_specs=[row_spec(D), row_spec(D), full_spec(D, D), full_spec(1, D)],
        out_specs=row_spec(D),
        out_shape=jax.ShapeDtypeStruct((Q, D), jnp.float32),
    )(sam, q2d, Wo.T[dh_perm], bo2)

    return out.reshape(1, Q, D)


# speedup vs baseline: 1.1064x; 1.0032x over previous
import functools

import jax
import jax.numpy as jnp
import numpy as np
from jax import lax
from jax.experimental import pallas as pl
from jax.experimental.pallas import tpu as pltpu
from jax.experimental.pallas import tpu_sc as plsc

NQB = 2
NH = 8
NP = 4
D = 256
DH = 32
SH = 128
SW = 128
Q = SH * SW
J = NP * NH * NQB
NSLOT = 4 * J

T1 = 2048

NWORK = 32
QPW = Q // NWORK
CQ = 8
NCHUNK = QPW // CQ
CROWS = CQ * NSLOT
OUTB = 8


def _prep_body(q_ref, rx_ref, ry_ref, wvt_ref, bv_ref, wsx_ref, bsx_ref,
               wsy_ref, bsy_ref, waw_ref, baw_ref, vt_ref, idx_ref, wgt_ref):
    q = q_ref[...]
    vt_ref[...] = (jnp.dot(q, wvt_ref[...], preferred_element_type=jnp.float32)
                   + bv_ref[...]).astype(jnp.bfloat16)
    sox = jnp.dot(q, wsx_ref[...], preferred_element_type=jnp.float32) + bsx_ref[...]
    soy = jnp.dot(q, wsy_ref[...], preferred_element_type=jnp.float32) + bsy_ref[...]
    aw = jnp.dot(q, waw_ref[...], preferred_element_type=jnp.float32) + baw_ref[...]

    a0, a1, a2, a3 = (aw[:, 0:16], aw[:, 16:32], aw[:, 32:48], aw[:, 48:64])
    m = jnp.maximum(jnp.maximum(a0, a1), jnp.maximum(a2, a3))
    e0, e1, e2, e3 = (jnp.exp(a0 - m), jnp.exp(a1 - m), jnp.exp(a2 - m),
                      jnp.exp(a3 - m))
    inv = 1.0 / (e0 + e1 + e2 + e3)
    w64 = jnp.concatenate([e0 * inv, e1 * inv, e2 * inv, e3 * inv], axis=1)

    rx64 = jnp.concatenate([rx_ref[...]] * 4, axis=1)
    ry64 = jnp.concatenate([ry_ref[...]] * 4, axis=1)

    locx = rx64 + sox * (1.0 / SW)
    locy = ry64 + soy * (1.0 / SH)
    gx = ((2.0 * locx - 1.0) + 1.0) * (SW / 2.0) - 0.5
    gy = ((2.0 * locy - 1.0) + 1.0) * (SH / 2.0) - 0.5
    x0 = jnp.floor(gx)
    y0 = jnp.floor(gy)
    wx1 = gx - x0
    wx0 = 1.0 - wx1
    wy1 = gy - y0
    wy0 = 1.0 - wy1

    hcol = (lax.broadcasted_iota(jnp.int32, (T1, J), 1) % 16) // 2
    corners = ((0.0, 0.0, wx0, wy0), (1.0, 0.0, wx1, wy0),
               (0.0, 1.0, wx0, wy1), (1.0, 1.0, wx1, wy1))
    for c, (dx, dy, wx, wy) in enumerate(corners):
        xi = x0 + dx
        yi = y0 + dy
        valid = ((xi >= 0.0) & (xi <= SW - 1.0)
                 & (yi >= 0.0) & (yi <= SH - 1.0))
        xc = jnp.clip(xi, 0.0, SW - 1.0).astype(jnp.int32)
        yc = jnp.clip(yi, 0.0, SH - 1.0).astype(jnp.int32)
        idx_ref[:, c * J:(c + 1) * J] = (yc * SW + xc) * NH + hcol
        wgt_ref[:, c * J:(c + 1) * J] = (wx * wy * w64 * 0.5
                                         * valid.astype(jnp.float32)
                                         ).astype(jnp.bfloat16)


def _out_body(s_ref, q_ref, wot_ref, bo_ref, o_ref):
    o_ref[...] = (jnp.dot(s_ref[...], wot_ref[...],
                          preferred_element_type=jnp.float32)
                  + bo_ref[...] + q_ref[...])


def _sc_body(vtab, idxh, wgth, outh,
             idx0, idx1, idx2, idx3, wgt0, wgt1, wgt2, wgt3,
             rows0, rows1, out_v,
             sga0, sga1,
             six0, six1, six2, six3, swg0, swg1, swg2, swg3, so):
    wid = lax.axis_index("s") * 2 + lax.axis_index("c")
    base = wid * (QPW * NSLOT)
    idxs = (idx0, idx1, idx2, idx3)
    wgts = (wgt0, wgt1, wgt2, wgt3)
    rows = (rows0, rows1)
    sgas = (sga0, sga1)
    sixs = (six0, six1, six2, six3)
    swgs = (swg0, swg1, swg2, swg3)

    def fetch(i, m):
        off = base + i * CROWS
        pltpu.async_copy(idxh.at[pl.ds(off, CROWS)], idxs[m], sixs[m])
        pltpu.async_copy(wgth.at[pl.ds(off, CROWS)], wgts[m], swgs[m])

    def fire(i, m, par):
        off = base + i * CROWS
        pltpu.make_async_copy(idxh.at[pl.ds(off, CROWS)], idxs[m], sixs[m]).wait()
        pltpu.make_async_copy(wgth.at[pl.ds(off, CROWS)], wgts[m], swgs[m]).wait()
        pltpu.async_copy(vtab.at[idxs[m]], rows[par], sgas[par])

    fetch(0, 0)
    fetch(1, 1)
    fetch(2, 2)
    fetch(3, 3)
    fire(0, 0, 0)
    fire(1, 1, 1)

    def quad(i4, carry):
        for m in range(4):
            par = m % 2
            i = i4 * 4 + m
            if m == 0:
                @pl.when((lax.rem(i4, 2) == 0) & (i4 >= 2))
                def _wait_out():
                    pltpu.make_async_copy(
                        out_v, outh.at[pl.ds(base, OUTB * CROWS)], so).wait()

            pltpu.make_async_copy(vtab.at[idxs[m]], rows[par], sgas[par]).wait()

            obase = lax.rem(i, OUTB) * CROWS

            def qloop(qi, qcarry):
                rbase = qi * NSLOT
                acc = [[jnp.zeros((16,), jnp.float32) for _ in range(2)]
                       for _ in range(NH)]
                for g in range(NSLOT // 32):
                    wv = wgts[m][pl.ds(rbase + g * 32, 32)]
                    we, wo = plsc.unpack(
                        wv, format=plsc.PackFormat.INTERLEAVED)
                    for k in range(32):
                        j = g * 32 + k
                        h = (j % 16) // 2
                        w = we[k // 2] if k % 2 == 0 else wo[k // 2]
                        row = rows[par][rbase + j]
                        re, ro = plsc.unpack(
                            row, format=plsc.PackFormat.INTERLEAVED)
                        acc[h][0] = acc[h][0] + w * re
                        acc[h][1] = acc[h][1] + w * ro
                for h in range(NH):
                    out_v[pl.ds(obase + rbase + h * DH, 16)] = acc[h][0]
                    out_v[pl.ds(obase + rbase + h * DH + 16, 16)] = acc[h][1]
                return qcarry

            lax.fori_loop(0, CQ, qloop, 0)

            @pl.when(i + 4 < NCHUNK)
            def _prefetch():
                fetch(i + 4, m)

            @pl.when(i + 2 < NCHUNK)
            def _fire():
                fire(i + 2, (m + 2) % 4, par)

            if m == 3:
                @pl.when(lax.rem(i4, 2) == 1)
                def _flush():
                    off0 = base + (i - (OUTB - 1)) * CROWS
                    pltpu.async_copy(out_v, outh.at[pl.ds(off0, OUTB * CROWS)], so)
        return carry

    lax.fori_loop(0, NCHUNK // 4, quad, 0)
    pltpu.make_async_copy(out_v, outh.at[pl.ds(base, OUTB * CROWS)], so).wait()


def kernel(query, key, value, reference_points, spatial_shapes,
           level_start_index, Wso, bso, Waw, baw, Wv, bv, Wo, bo):
    del key, value, spatial_shapes, level_start_index
    q2d = query.reshape(Q, D)

    Wso_c = Wso[:, :D] + Wso[:, D:]
    Waw_c = Waw[:, :D] + Waw[:, D:]

    so_x_rows, so_y_rows, aw_rows = [], [], []
    for p in range(NP):
        for h in range(NH):
            for bq in range(NQB):
                so_x_rows.append(h * 16 + bq * 8 + p * 2 + 0)
                so_y_rows.append(h * 16 + bq * 8 + p * 2 + 1)
                aw_rows.append(h * 8 + bq * 4 + p)
    so_x_rows = np.array(so_x_rows)
    so_y_rows = np.array(so_y_rows)
    aw_rows = np.array(aw_rows)

    wvt = Wv.T
    wsx = Wso_c[so_x_rows].T
    wsy = Wso_c[so_y_rows].T
    waw = Waw_c[aw_rows].T
    bsx = bso[so_x_rows].reshape(1, J)
    bsy = bso[so_y_rows].reshape(1, J)
    bawp = baw[aw_rows].reshape(1, J)
    bv2 = bv.reshape(1, D)
    bo2 = bo.reshape(1, D)

    rx = reference_points[:, :, 0, 0]
    ry = reference_points[:, :, 0, 1]
    rx16 = jnp.tile(rx.T, (1, 8))
    ry16 = jnp.tile(ry.T, (1, 8))

    dh_perm = np.empty((D,), np.int64)
    for h in range(NH):
        for k in range(16):
            dh_perm[h * DH + k] = h * DH + 2 * k
            dh_perm[h * DH + 16 + k] = h * DH + 2 * k + 1

    grid = (Q // T1,)
    row_spec = lambda w: pl.BlockSpec((T1, w), lambda i: (i, 0))
    full_spec = lambda a, b: pl.BlockSpec((a, b), lambda i: (0, 0))

    vt, idx, wgt = pl.pallas_call(
        _prep_body,
        grid=grid,
        in_specs=[row_spec(D), row_spec(16), row_spec(16),
                  full_spec(D, D), full_spec(1, D),
                  full_spec(D, J), full_spec(1, J),
                  full_spec(D, J), full_spec(1, J),
                  full_spec(D, J), full_spec(1, J)],
        out_specs=[row_spec(D), row_spec(NSLOT), row_spec(NSLOT)],
        out_shape=[jax.ShapeDtypeStruct((Q, D), jnp.bfloat16),
                   jax.ShapeDtypeStruct((Q, NSLOT), jnp.int32),
                   jax.ShapeDtypeStruct((Q, NSLOT), jnp.bfloat16)],
    )(q2d, rx16, ry16, wvt, bv2, wsx, bsx, wsy, bsy, waw, bawp)

    mesh = plsc.VectorSubcoreMesh(core_axis_name="c", subcore_axis_name="s")
    sam_flat = pl.kernel(
        _sc_body,
        mesh=mesh,
        compiler_params=pltpu.CompilerParams(use_tc_tiling_on_sc=False,
                                             needs_layout_passes=False),
        out_type=jax.ShapeDtypeStruct((Q * D,), jnp.float32),
        scratch_types=[
            pltpu.VMEM((CROWS,), jnp.int32),
            pltpu.VMEM((CROWS,), jnp.int32),
            pltpu.VMEM((CROWS,), jnp.int32),
            pltpu.VMEM((CROWS,), jnp.int32),
            pltpu.VMEM((CROWS,), jnp.bfloat16),
            pltpu.VMEM((CROWS,), jnp.bfloat16),
            pltpu.VMEM((CROWS,), jnp.bfloat16),
            pltpu.VMEM((CROWS,), jnp.bfloat16),
            pltpu.VMEM((CROWS, DH), jnp.bfloat16),
            pltpu.VMEM((CROWS, DH), jnp.bfloat16),
            pltpu.VMEM((OUTB * CROWS,), jnp.float32),
            pltpu.SemaphoreType.DMA,
            pltpu.SemaphoreType.DMA,
            pltpu.SemaphoreType.DMA,
            pltpu.SemaphoreType.DMA,
            pltpu.SemaphoreType.DMA,
            pltpu.SemaphoreType.DMA,
            pltpu.SemaphoreType.DMA,
            pltpu.SemaphoreType.DMA,
            pltpu.SemaphoreType.DMA,
            pltpu.SemaphoreType.DMA,
            pltpu.SemaphoreType.DMA,
        ],
    )(vt.reshape(Q * NH, DH), idx.reshape(Q * NSLOT), wgt.reshape(Q * NSLOT))

    sam = sam_flat.reshape(Q, D)

    out = pl.pallas_call(
        _out_body,
        grid=grid,
        in_specs=[row_spec(D), row_spec(D), full_spec(D, D), full_spec(1, D)],
        out_specs=row_spec(D),
        out_shape=jax.ShapeDtypeStruct((Q, D), jnp.float32),
    )(sam, q2d, Wo.T[dh_perm], bo2)

    return out.reshape(1, Q, D)
